# Initial kernel scaffold; baseline (speedup 1.0000x reference)
import jax
import jax.numpy as jnp
from jax.experimental import pallas as pl


def kernel(x, edge_index, edge_type, W, W_self):
    raise NotImplementedError("write your pallas kernel here")


# speedup vs baseline: 2.7285x; 2.7285x over previous
import functools

import jax
import jax.numpy as jnp
from jax import lax
from jax.experimental import pallas as pl
from jax.experimental.pallas import tpu as pltpu
from jax.experimental.pallas import tpu_sc as plsc

N = 10000
E = 320000
D = 128
R = 16

NC = 2
NS = 16
NW = NC * NS
C = 128
NCHUNKS = E // C
NPAD = 10240
ROWS_PER_TILE = NPAD // NS
ZROWS = 128

BN = 1000


def _matmul_body(h_ref, w_ref, z_ref):
    z_ref[0] = jnp.dot(h_ref[...], w_ref[0], preferred_element_type=jnp.float32)


def _tc_transform(h, wcat):
    nt = N // BN
    rp = R + 1
    return pl.pallas_call(
        _matmul_body,
        grid=(nt, rp),
        in_specs=[
            pl.BlockSpec((BN, D), lambda i, r: (i, 0)),
            pl.BlockSpec((1, D, D), lambda i, r: (r, 0, 0)),
        ],
        out_specs=pl.BlockSpec((1, BN, D), lambda i, r: (r, i, 0)),
        out_shape=jax.ShapeDtypeStruct((rp, N, D), jnp.float32),
    )(h, wcat)


def _fill_rows(ref, nrows, val):
    def _row(i, carry):
        for j in range(D // 16):
            ref[i, pl.ds(16 * j, 16)] = jnp.full((16,), val, jnp.float32)
        return carry

    lax.fori_loop(0, nrows, _row, 0)


def _zero_shared(acc_sh, rows_v, s):
    for k in range(ROWS_PER_TILE // ZROWS):
        pltpu.sync_copy(
            rows_v.at[pl.ds(0, ZROWS)],
            acc_sh.at[pl.ds(s * ROWS_PER_TILE + k * ZROWS, ZROWS)],
        )


def _writeback(acc_sh, rows_v, acc_out, c, s):
    row0 = s * ROWS_PER_TILE
    for k in range(ROWS_PER_TILE // ZROWS):
        r0 = row0 + k * ZROWS
        pltpu.sync_copy(acc_sh.at[pl.ds(r0, ZROWS)], rows_v)
        pltpu.sync_copy(rows_v, acc_out.at[c, pl.ds(r0, ZROWS)])


def _sc_body(z_ref, src_ref, dst_ref, et_ref, acc_out,
             src_v, et_v, dst_v, rid_v, rows_v, acc_sh, sem):
    c = lax.axis_index("c")
    s = lax.axis_index("s")
    wid = s * NC + c

    _fill_rows(rows_v, C, 0.0)
    _zero_shared(acc_sh, rows_v, s)
    plsc.subcore_barrier()

    ntrips = (NCHUNKS - 1 - wid) // NW + 1

    def _chunk(i, carry):
        base = (wid + i * NW) * C
        pltpu.sync_copy(src_ref.at[pl.ds(base, C)], src_v)
        pltpu.sync_copy(et_ref.at[pl.ds(base, C)], et_v)
        pltpu.sync_copy(dst_ref.at[pl.ds(base, C)], dst_v)
        for j in range(C // 16):
            sl = pl.ds(16 * j, 16)
            rid_v[sl] = et_v[sl] * N + src_v[sl]
        pltpu.async_copy(z_ref.at[rid_v], rows_v, sem).wait()
        pltpu.sync_copy(rows_v, acc_sh.at[dst_v], add=True)
        return carry

    lax.fori_loop(0, ntrips, _chunk, 0)

    plsc.subcore_barrier()
    _writeback(acc_sh, rows_v, acc_out, c, s)


def _deg_body(dst_ref, deg_out, dst_v, rows_v, deg_sh, sem):
    c = lax.axis_index("c")
    s = lax.axis_index("s")
    wid = s * NC + c

    _fill_rows(rows_v, C, 0.0)
    _zero_shared(deg_sh, rows_v, s)
    _fill_rows(rows_v, C, 1.0)
    plsc.subcore_barrier()

    ntrips = (NCHUNKS - 1 - wid) // NW + 1

    def _chunk(i, carry):
        base = (wid + i * NW) * C
        pltpu.sync_copy(dst_ref.at[pl.ds(base, C)], dst_v)
        pltpu.sync_copy(rows_v, deg_sh.at[dst_v], add=True)
        return carry

    lax.fori_loop(0, ntrips, _chunk, 0)

    plsc.subcore_barrier()
    _writeback(deg_sh, rows_v, deg_out, c, s)


def _sc_mesh():
    return plsc.VectorSubcoreMesh(
        core_axis_name="c", subcore_axis_name="s",
        num_cores=NC, num_subcores=NS,
    )


@functools.lru_cache(maxsize=None)
def _make_sc():
    return pl.kernel(
        _sc_body,
        out_type=[jax.ShapeDtypeStruct((NC, NPAD, D), jnp.float32)],
        mesh=_sc_mesh(),
        scratch_types=[
            pltpu.VMEM((C,), jnp.int32),
            pltpu.VMEM((C,), jnp.int32),
            pltpu.VMEM((C,), jnp.int32),
            pltpu.VMEM((C,), jnp.int32),
            pltpu.VMEM((C, D), jnp.float32),
            pltpu.VMEM_SHARED((NPAD, D), jnp.float32),
            pltpu.SemaphoreType.DMA,
        ],
    )


@functools.lru_cache(maxsize=None)
def _make_deg():
    return pl.kernel(
        _deg_body,
        out_type=[jax.ShapeDtypeStruct((NC, NPAD, D), jnp.float32)],
        mesh=_sc_mesh(),
        scratch_types=[
            pltpu.VMEM((C,), jnp.int32),
            pltpu.VMEM((C, D), jnp.float32),
            pltpu.VMEM_SHARED((NPAD, D), jnp.float32),
            pltpu.SemaphoreType.DMA,
        ],
    )


def _combine_body(acc_ref, degp_ref, z_ref, out_ref):
    d = jnp.maximum(degp_ref[0, :, 0:1] + degp_ref[1, :, 0:1], 1.0)
    agg = acc_ref[0] + acc_ref[1]
    out_ref[...] = jnp.maximum(agg / d + z_ref[0], 0.0)


def _combine(acc, degp, z):
    nt = N // BN
    return pl.pallas_call(
        _combine_body,
        grid=(nt,),
        in_specs=[
            pl.BlockSpec((NC, BN, D), lambda i: (0, i, 0)),
            pl.BlockSpec((NC, BN, D), lambda i: (0, i, 0)),
            pl.BlockSpec((1, BN, D), lambda i: (R, i, 0)),
        ],
        out_specs=pl.BlockSpec((BN, D), lambda i: (i, 0)),
        out_shape=jax.ShapeDtypeStruct((N, D), jnp.float32),
    )(acc, degp, z)


def kernel(x, edge_index, edge_type, W, W_self):
    src = edge_index[0]
    dst = edge_index[1]
    et = edge_type
    (degp,) = _make_deg()(dst)
    h = x
    for l in range(2):
        wcat = jnp.concatenate([W[l], W_self[l][None]], axis=0)
        z = _tc_transform(h, wcat)
        zflat = z.reshape(((R + 1) * N, D))
        (acc,) = _make_sc()(zflat, src, dst, et)
        h = _combine(acc, degp, z)
    return h
